# baseline (speedup 1.0000x reference)
import functools

import jax
import jax.numpy as jnp
from jax import lax
from jax.experimental import pallas as pl
from jax.experimental.pallas import tpu as pltpu
from jax.experimental.pallas import tpu_sc as plsc

_K = 8192
_D = 32
_EPS = 1e-5
_BETA = 0.2
_ROWS = 4096
_BLK = 128


def _conv2d(x, w, b):
    out = lax.conv_general_dilated(
        x.astype(jnp.bfloat16), w.astype(jnp.bfloat16),
        window_strides=(2, 2), padding=((1, 1), (1, 1)),
        dimension_numbers=('NCHW', 'OIHW', 'NCHW'),
        preferred_element_type=jnp.float32)
    return out + b[None, :, None, None]


def _conv_transpose2d(x, w, b):
    w_f = jnp.flip(w, axis=(2, 3))
    w_t = jnp.transpose(w_f, (1, 0, 2, 3))
    out = lax.conv_general_dilated(
        x, w_t, window_strides=(1, 1), padding=((2, 2), (2, 2)),
        lhs_dilation=(2, 2), dimension_numbers=('NCHW', 'OIHW', 'NCHW'))
    return out + b[None, :, None, None]


def _batchnorm(x, g, b):
    m = jnp.mean(x, axis=(0, 2, 3), keepdims=True)
    v = jnp.var(x, axis=(0, 2, 3), keepdims=True)
    xn = (x - m) / jnp.sqrt(v + _EPS)
    return xn * g[None, :, None, None] + b[None, :, None, None]


_BLK = 256
_KC = 2048


def _vq_body(q_ref, cb_ref, idx_ref, msum_ref, e2_ref, cbb_ref):
    i = pl.program_id(0)

    @pl.when(i == 0)
    def _():
        cb = cb_ref[...]
        e2_ref[...] = jnp.sum(cb * cb, axis=1)[None, :]
        cbb_ref[...] = cb.astype(jnp.bfloat16)
        msum_ref[0, 0] = 0.0

    q = q_ref[...]
    qb = q.astype(jnp.bfloat16)
    q2 = jnp.sum(q * q, axis=1, keepdims=True)

    def chunk(k, carry):
        run_min, run_idx = carry
        cb_c = cbb_ref[pl.ds(k * _KC, _KC), :]
        cross = lax.dot_general(
            qb, cb_c, dimension_numbers=(((1,), (1,)), ((), ())),
            preferred_element_type=jnp.float32)
        scores = (q2 + e2_ref[:, pl.ds(k * _KC, _KC)]) - 2.0 * cross
        m_c = jnp.min(scores, axis=1)
        kio = lax.broadcasted_iota(jnp.int32, scores.shape, 1) + k * _KC
        idx_c = jnp.min(jnp.where(scores == m_c[:, None], kio, _K), axis=1)
        upd = m_c < run_min
        return jnp.minimum(run_min, m_c), jnp.where(upd, idx_c, run_idx)

    run_min, run_idx = lax.fori_loop(
        0, _K // _KC, chunk,
        (jnp.full((_BLK,), jnp.inf, jnp.float32), jnp.zeros((_BLK,), jnp.int32)))
    idx_ref[0, 0, :] = run_idx
    msum_ref[0, 0] += jnp.sum(jnp.maximum(run_min, 0.0))


def _vq_argmin(q, cb):
    nblk = _ROWS // _BLK
    idx3, msum = pl.pallas_call(
        _vq_body,
        grid=(nblk,),
        in_specs=[
            pl.BlockSpec((_BLK, _D), lambda i: (i, 0)),
            pl.BlockSpec((_K, _D), lambda i: (0, 0)),
        ],
        out_specs=[
            pl.BlockSpec((1, 1, _BLK), lambda i: (i, 0, 0)),
            pl.BlockSpec(memory_space=pltpu.SMEM),
        ],
        out_shape=[
            jax.ShapeDtypeStruct((nblk, 1, _BLK), jnp.int32),
            jax.ShapeDtypeStruct((1, 1), jnp.float32),
        ],
        scratch_shapes=[
            pltpu.VMEM((1, _K), jnp.float32),
            pltpu.VMEM((_K, _D), jnp.bfloat16),
        ],
    )(q, cb)
    return idx3.reshape(_ROWS), msum[0, 0]


@functools.lru_cache(maxsize=1)
def _make_sc_gather():
    info = plsc.get_sparse_core_info()
    nw = info.num_cores * info.num_subcores
    b_per_w = _ROWS // nw
    mesh = plsc.VectorSubcoreMesh(core_axis_name="c", subcore_axis_name="s")

    @functools.partial(
        pl.kernel, mesh=mesh,
        out_type=jax.ShapeDtypeStruct((_ROWS, _D), jnp.float32),
        scratch_types=[
            pltpu.VMEM((b_per_w,), jnp.int32),
            pltpu.VMEM((b_per_w, _D), jnp.float32),
            pltpu.SemaphoreType.DMA,
        ],
        compiler_params=pltpu.CompilerParams(use_tc_tiling_on_sc=False),
    )
    def gather(table_hbm, idx_hbm, out_hbm, idx_v, rows_v, sem):
        wid = lax.axis_index("s") * info.num_cores + lax.axis_index("c")
        base = wid * b_per_w
        pltpu.sync_copy(idx_hbm.at[pl.ds(base, b_per_w)], idx_v)
        pltpu.async_copy(table_hbm.at[idx_v], rows_v, sem).wait()
        pltpu.sync_copy(rows_v, out_hbm.at[pl.ds(base, b_per_w)])

    return gather


def _gather_rows(table, idx):
    return _make_sc_gather()(table, idx)


def _taps(w):
    ci, co = w.shape[0], w.shape[1]
    mats = []
    for a in range(3):
        for b in range(3):
            cols = []
            for py in range(2):
                for px in range(2):
                    ky, kx = py + 3 - 2 * a, px + 3 - 2 * b
                    if 0 <= ky < 4 and 0 <= kx < 4:
                        cols.append(w[:, :, ky, kx])
                    else:
                        cols.append(jnp.zeros((ci, co), w.dtype))
            mats.append(jnp.concatenate(cols, axis=1).T)
    return jnp.stack(mats).astype(jnp.bfloat16)


def _deconv_body(hw, wp, x_ref, v_ref, out_ref, stats_ref):
    b = pl.program_id(0)
    nrow = hw * wp
    xb = x_ref[0].astype(jnp.bfloat16)
    acc = lax.dot_general(
        v_ref[0], xb[:, 0:nrow],
        dimension_numbers=(((1,), (0,)), ((), ())),
        preferred_element_type=jnp.float32)
    for t in range(1, 9):
        a, bb = divmod(t, 3)
        off = a * wp + bb
        acc += lax.dot_general(
            v_ref[t], xb[:, off:off + nrow],
            dimension_numbers=(((1,), (0,)), ((), ())),
            preferred_element_type=jnp.float32)
    out_ref[0] = acc
    r = lax.broadcasted_iota(jnp.int32, (1, nrow), 1)
    am = jnp.where((r % wp) < (wp - 2), acc, 0.0)
    s1 = jnp.sum(am, axis=1)
    s2 = jnp.sum(am * am, axis=1)

    @pl.when(b == 0)
    def _():
        stats_ref[...] = jnp.zeros_like(stats_ref)

    stats_ref[0, :] += s1
    stats_ref[1, :] += s2


def _deconv(xflat, v, h, w):
    B = xflat.shape[0]
    co4, ci = v.shape[1], v.shape[2]
    wp = w + 2
    out6, stats = pl.pallas_call(
        functools.partial(_deconv_body, h, wp),
        grid=(B,),
        in_specs=[
            pl.BlockSpec((1, ci, (h + 3) * wp), lambda i: (i, 0, 0)),
            pl.BlockSpec((9, co4, ci), lambda i: (0, 0, 0)),
        ],
        out_specs=[
            pl.BlockSpec((1, co4, h * wp), lambda i: (i, 0, 0)),
            pl.BlockSpec((2, co4), lambda i: (0, 0)),
        ],
        out_shape=[
            jax.ShapeDtypeStruct((B, co4, h * wp), jnp.float32),
            jax.ShapeDtypeStruct((2, co4), jnp.float32),
        ],
    )(xflat, v)
    return out6, stats


def _bn_coeffs(stats, g, beta, n):
    co = g.shape[0]
    s1 = stats[0].reshape(4, co).sum(0)
    s2 = stats[1].reshape(4, co).sum(0)
    m = s1 / n
    var = s2 / n - m * m
    scale = g / jnp.sqrt(var + _EPS)
    shift = beta - m * scale
    return scale, shift


def _interleave(out6, B, h, w, co):
    wp = w + 2
    planes = out6.reshape(B, 2, 2, co, h, wp)[..., :w]
    img = None
    for py in range(2):
        for px in range(2):
            p = lax.pad(planes[:, py, px], 0.0,
                        ((0, 0, 0), (0, 0, 0), (py, 1 - py, 1), (px, 1 - px, 1)))
            img = p if img is None else img + p
    return img


def _pad_flat(img):
    B, co = img.shape[0], img.shape[1]
    h, w = img.shape[2], img.shape[3]
    return jnp.pad(img, ((0, 0), (0, 0), (1, 2), (1, 1))).reshape(B, co, (h + 3) * (w + 2))


def _final_body(img_ref, x_ref, sc_ref, out_ref, sse_ref):
    scale = sc_ref[0, 0]
    shift = sc_ref[0, 1]
    out = img_ref[...] * scale + shift
    out_ref[...] = out
    d = x_ref[...] - out
    sse_ref[0, 0] = jnp.sum(d * d)


def _final(img, x, scale, shift):
    sc = jnp.stack([scale, shift]).reshape(1, 2)
    zeros = (0,) * len(img.shape)
    out, sse = pl.pallas_call(
        _final_body,
        in_specs=[
            pl.BlockSpec(img.shape, lambda: zeros),
            pl.BlockSpec(x.shape, lambda: zeros),
            pl.BlockSpec(memory_space=pltpu.SMEM),
        ],
        out_specs=[
            pl.BlockSpec(img.shape, lambda: zeros),
            pl.BlockSpec(memory_space=pltpu.SMEM),
        ],
        out_shape=[
            jax.ShapeDtypeStruct(img.shape, jnp.float32),
            jax.ShapeDtypeStruct((1, 1), jnp.float32),
        ],
    )(img, x, sc)
    return out, sse[0, 0]


def kernel(x, ew1, eb1, eg1, ebt1, ew2, eb2, eg2, ebt2, ew3, eb3, eg3, ebt3,
           codebook, dw1, db1, dg1, dbt1, dw2, db2, dg2, dbt2, dw3, db3, dg3, dbt3):
    h = jax.nn.relu(_batchnorm(_conv2d(x, ew1, eb1), eg1, ebt1))
    h = jax.nn.relu(_batchnorm(_conv2d(h, ew2, eb2), eg2, ebt2))
    quant_input = jax.nn.relu(_batchnorm(_conv2d(h, ew3, eb3), eg3, ebt3))
    B, C, H, W = quant_input.shape
    q = jnp.transpose(quant_input, (0, 2, 3, 1)).reshape(B * H * W, C)

    idx, msum = _vq_argmin(q, codebook)

    rows = _gather_rows(codebook, idx)

    mse = msum / (B * H * W * C)
    qloss = (1.0 + _BETA) * mse

    quant_chw = jnp.transpose(rows.reshape(B, H, W, C), (0, 3, 1, 2))
    out6, st = _deconv(_pad_flat(quant_chw), _taps(dw1), H, W)
    sc1, sh1 = _bn_coeffs(st, dg1, dbt1, B * 4 * H * W)
    img = jax.nn.relu(_interleave(out6, B, H, W, 16)
                      * sc1[None, :, None, None] + sh1[None, :, None, None])

    h2, w2 = 2 * H, 2 * W
    out6, st = _deconv(_pad_flat(img), _taps(dw2), h2, w2)
    sc2, sh2 = _bn_coeffs(st, dg2, dbt2, B * 4 * h2 * w2)
    img = jax.nn.relu(_interleave(out6, B, h2, w2, 16)
                      * sc2[None, :, None, None] + sh2[None, :, None, None])

    h3, w3 = 2 * h2, 2 * w2
    out6, st = _deconv(_pad_flat(img), _taps(dw3), h3, w3)
    sc3, sh3 = _bn_coeffs(st, dg3, dbt3, B * 4 * h3 * w3)
    img_raw = _interleave(out6, B, h3, w3, 1)

    output, sse = _final(img_raw, x, sc3[0], sh3[0])

    reconstruction_loss = sse / (B * 4 * h3 * w3)
    total_loss = qloss + reconstruction_loss
    enc_idx = idx.reshape(B, H, W)
    return (output, total_loss, enc_idx)


# speedup vs baseline: 1.7022x; 1.7022x over previous
import functools

import jax
import jax.numpy as jnp
from jax import lax
from jax.experimental import pallas as pl
from jax.experimental.pallas import tpu as pltpu
from jax.experimental.pallas import tpu_sc as plsc

_K = 8192
_D = 32
_EPS = 1e-5
_BETA = 0.2
_ROWS = 4096
_BLK = 128


def _conv2d(x, w, b):
    out = lax.conv_general_dilated(
        x, w, window_strides=(2, 2), padding=((1, 1), (1, 1)),
        dimension_numbers=('NCHW', 'OIHW', 'NCHW'))
    return out + b[None, :, None, None]


def _conv_transpose2d(x, w, b):
    w_f = jnp.flip(w, axis=(2, 3))
    w_t = jnp.transpose(w_f, (1, 0, 2, 3))
    out = lax.conv_general_dilated(
        x, w_t, window_strides=(1, 1), padding=((2, 2), (2, 2)),
        lhs_dilation=(2, 2), dimension_numbers=('NCHW', 'OIHW', 'NCHW'))
    return out + b[None, :, None, None]


def _batchnorm(x, g, b):
    m = jnp.mean(x, axis=(0, 2, 3), keepdims=True)
    v = jnp.var(x, axis=(0, 2, 3), keepdims=True)
    xn = (x - m) / jnp.sqrt(v + _EPS)
    return xn * g[None, :, None, None] + b[None, :, None, None]


_BLK = 256
_KC = 2048


def _vq_body(qb_ref, q2_ref, cbb_ref, e2_ref, idx_ref, msum_ref):
    i = pl.program_id(0)

    @pl.when(i == 0)
    def _():
        msum_ref[0, 0] = 0.0

    qb = qb_ref[...]
    q2 = q2_ref[0, 0, :][:, None]
    run_min = jnp.full((_BLK,), jnp.inf, jnp.float32)
    run_idx = jnp.zeros((_BLK,), jnp.int32)
    for k in range(_K // _KC):
        cb_c = cbb_ref[k * _KC:(k + 1) * _KC, :]
        cross = lax.dot_general(
            qb, cb_c, dimension_numbers=(((1,), (1,)), ((), ())),
            precision=lax.Precision.DEFAULT,
            preferred_element_type=jnp.float32)
        scores = (q2 + e2_ref[:, k * _KC:(k + 1) * _KC]) - 2.0 * cross
        m_c = jnp.min(scores, axis=1)
        kio = lax.broadcasted_iota(jnp.int32, scores.shape, 1) + k * _KC
        idx_c = jnp.min(jnp.where(scores == m_c[:, None], kio, _K), axis=1)
        upd = m_c < run_min
        run_idx = jnp.where(upd, idx_c, run_idx)
        run_min = jnp.minimum(run_min, m_c)
    idx_ref[0, 0, :] = run_idx
    msum_ref[0, 0] += jnp.sum(jnp.maximum(run_min, 0.0))


def _vq_argmin(qb, q2, cbb, e2):
    nblk = _ROWS // _BLK
    idx3, msum = pl.pallas_call(
        _vq_body,
        grid=(nblk,),
        in_specs=[
            pl.BlockSpec((_BLK, _D), lambda i: (i, 0)),
            pl.BlockSpec((1, 1, _BLK), lambda i: (i, 0, 0)),
            pl.BlockSpec((_K, _D), lambda i: (0, 0)),
            pl.BlockSpec((1, _K), lambda i: (0, 0)),
        ],
        out_specs=[
            pl.BlockSpec((1, 1, _BLK), lambda i: (i, 0, 0)),
            pl.BlockSpec(memory_space=pltpu.SMEM),
        ],
        out_shape=[
            jax.ShapeDtypeStruct((nblk, 1, _BLK), jnp.int32),
            jax.ShapeDtypeStruct((1, 1), jnp.float32),
        ],
    )(qb, q2, cbb, e2)
    return idx3.reshape(_ROWS), msum[0, 0]


@functools.lru_cache(maxsize=1)
def _make_sc_gather():
    info = plsc.get_sparse_core_info()
    nw = info.num_cores * info.num_subcores
    b_per_w = _ROWS // nw
    mesh = plsc.VectorSubcoreMesh(core_axis_name="c", subcore_axis_name="s")

    @functools.partial(
        pl.kernel, mesh=mesh,
        out_type=jax.ShapeDtypeStruct((_ROWS, _D), jnp.float32),
        scratch_types=[
            pltpu.VMEM((b_per_w,), jnp.int32),
            pltpu.VMEM((b_per_w, _D), jnp.float32),
            pltpu.SemaphoreType.DMA,
        ],
        compiler_params=pltpu.CompilerParams(use_tc_tiling_on_sc=False),
    )
    def gather(table_hbm, idx_hbm, out_hbm, idx_v, rows_v, sem):
        wid = lax.axis_index("s") * info.num_cores + lax.axis_index("c")
        base = wid * b_per_w
        pltpu.sync_copy(idx_hbm.at[pl.ds(base, b_per_w)], idx_v)
        pltpu.async_copy(table_hbm.at[idx_v], rows_v, sem).wait()
        pltpu.sync_copy(rows_v, out_hbm.at[pl.ds(base, b_per_w)])

    return gather


def _gather_rows(table, idx):
    return _make_sc_gather()(table, idx)


def kernel(x, ew1, eb1, eg1, ebt1, ew2, eb2, eg2, ebt2, ew3, eb3, eg3, ebt3,
           codebook, dw1, db1, dg1, dbt1, dw2, db2, dg2, dbt2, dw3, db3, dg3, dbt3):
    h = jax.nn.relu(_batchnorm(_conv2d(x, ew1, eb1), eg1, ebt1))
    h = jax.nn.relu(_batchnorm(_conv2d(h, ew2, eb2), eg2, ebt2))
    quant_input = jax.nn.relu(_batchnorm(_conv2d(h, ew3, eb3), eg3, ebt3))
    B, C, H, W = quant_input.shape
    q3 = jnp.transpose(quant_input, (0, 2, 3, 1)).reshape(B, H * W, C)
    q2 = jnp.sum(q3 * q3, axis=-1, keepdims=True)
    e2 = jnp.sum(codebook * codebook, axis=-1)
    q = q3.reshape(B * H * W, C)

    nblk = _ROWS // _BLK
    idx, msum = _vq_argmin(
        q, q2.reshape(nblk, 1, _BLK), codebook, e2[None, :])

    rows = _gather_rows(codebook, idx)

    mse = msum / (B * H * W * C)
    qloss = (1.0 + _BETA) * mse

    quant_nchw = jnp.transpose(rows.reshape(B, H, W, C), (0, 3, 1, 2))
    d = jax.nn.relu(_batchnorm(_conv_transpose2d(quant_nchw, dw1, db1), dg1, dbt1))
    d = jax.nn.relu(_batchnorm(_conv_transpose2d(d, dw2, db2), dg2, dbt2))
    output = _batchnorm(_conv_transpose2d(d, dw3, db3), dg3, dbt3)

    reconstruction_loss = jnp.mean((x - output) ** 2)
    total_loss = qloss + reconstruction_loss
    enc_idx = idx.reshape(B, H, W)
    return (output, total_loss, enc_idx)
